# baseline (speedup 1.0000x reference)
import functools

import jax
import jax.numpy as jnp
from jax import lax
from jax.experimental import pallas as pl
from jax.experimental.pallas import tpu as pltpu
from jax.experimental.pallas import tpu_sc as plsc

G = 4096
N = 100000
D = 128
NB = 8
HALF = D // 2

NC = 2
NS = 16
NW = NC * NS

CH = 80
NCHUNK = N // CH
AITERS = -(-NCHUNK // NW)
CITERS = -(-NCHUNK // NS)
GPT = G // NS


def _embed_body(tau_ref, emb_ref, alpha_ref, sigma_ref):
    t = tau_ref[...]
    k = lax.broadcasted_iota(jnp.int32, (1, HALF), 1).astype(jnp.float32)
    freqs = jnp.exp((-jnp.log(10000.0) / HALF) * k)
    args = t * freqs
    emb_ref[:, :HALF] = jnp.sin(args)
    emb_ref[:, HALF:] = jnp.cos(args)
    alpha_ref[...] = 1.0 - t
    sigma_ref[...] = t


_embed = pl.pallas_call(
    _embed_body,
    out_shape=(
        jax.ShapeDtypeStruct((G, D), jnp.float32),
        jax.ShapeDtypeStruct((G, 1), jnp.float32),
        jax.ShapeDtypeStruct((G, 1), jnp.float32),
    ),
)


_mesh = plsc.VectorSubcoreMesh(
    core_axis_name="c", subcore_axis_name="s", num_cores=NC, num_subcores=NS
)


@functools.partial(
    pl.kernel,
    out_type=(
        jax.ShapeDtypeStruct((N, D), jnp.float32),
        jax.ShapeDtypeStruct((G, NB), jnp.float32),
    ),
    mesh=_mesh,
    compiler_params=pltpu.CompilerParams(needs_layout_passes=False),
    scratch_types=(
        pltpu.VMEM((CH,), jnp.int32),
        pltpu.VMEM((CH, D), jnp.float32),
        pltpu.VMEM((96,), jnp.int32),
        pltpu.VMEM((G,), jnp.int32),
        pltpu.VMEM((NS * GPT,), jnp.int32),
        pltpu.VMEM((GPT,), jnp.int32),
        pltpu.VMEM((GPT, NB), jnp.float32),
        pltpu.VMEM_SHARED((NS, G), jnp.int32),
        pltpu.SemaphoreType.DMA,
    ),
)
def _sc_body(emb_hbm, batch_hbm, cond_hbm, bits_hbm,
             idx_v, rows_v, ext_v, pcount_v, ptmp_v, csum_v, bits_v,
             shared, sem):
    cid = lax.axis_index("c")
    sid = lax.axis_index("s")
    wid = sid * NC + cid

    zeros16 = jnp.zeros((16,), jnp.int32)
    iota16 = lax.iota(jnp.int32, 16)

    @pl.when(cid == 0)
    def _counts():
        def zero_body(i, _):
            pcount_v[pl.ds(i * 16, 16)] = zeros16
            return _
        lax.fori_loop(0, G // 16, zero_body, None)

        def count_chunk(i, _):
            c = i * NS + sid

            @pl.when(c < NCHUNK)
            def _():
                base = c * CH
                last = c == NCHUNK - 1

                @pl.when(jnp.logical_not(last))
                def _():
                    pltpu.sync_copy(batch_hbm.at[pl.ds(base, 88)],
                                    ext_v.at[pl.ds(0, 88)])

                @pl.when(last)
                def _():
                    pltpu.sync_copy(batch_hbm.at[pl.ds(base, CH)],
                                    ext_v.at[pl.ds(0, CH)])
                    ext_v[pl.ds(CH, 16)] = zeros16 - 1

                for j0 in range(0, CH, 16):
                    cur = ext_v[pl.ds(j0, 16)]
                    nxt = ext_v[pl.ds(j0 + 1, 16)]
                    m = cur != nxt
                    gi = (base + j0 + 1) + iota16
                    plsc.addupdate_scatter(pcount_v, [cur], gi, mask=m)
                    plsc.addupdate_scatter(pcount_v, [nxt], zeros16 - gi,
                                           mask=m & (nxt >= 0))
            return _
        lax.fori_loop(0, CITERS, count_chunk, None)

        pltpu.sync_copy(pcount_v, shared.at[sid])
        plsc.subcore_barrier()

    def gather_chunk(i, _):
        c = i * NW + wid

        @pl.when(c < NCHUNK)
        def _():
            base = c * CH
            pltpu.sync_copy(batch_hbm.at[pl.ds(base, CH)], idx_v)
            pltpu.async_copy(emb_hbm.at[idx_v], rows_v, sem).wait()
            pltpu.sync_copy(rows_v, cond_hbm.at[pl.ds(base, CH)])
        return _
    lax.fori_loop(0, AITERS, gather_chunk, None)

    @pl.when(cid == 0)
    def _bits():
        g0 = sid * GPT
        for p in range(NS):
            pltpu.sync_copy(shared.at[p, pl.ds(g0, GPT)],
                            ptmp_v.at[pl.ds(p * GPT, GPT)])
        for v in range(0, GPT, 16):
            acc = zeros16
            for p in range(NS):
                acc = acc + ptmp_v[pl.ds(p * GPT + v, 16)]
            csum_v[pl.ds(v, 16)] = acc
        for v in range(0, GPT, 16):
            cnt = csum_v[pl.ds(v, 16)]
            rows = v + iota16
            for b in range(NB):
                bit = ((cnt >> b) & 1).astype(jnp.float32)
                cols = jnp.full((16,), b, jnp.int32)
                plsc.store_scatter(bits_v, [rows, cols], bit)
        pltpu.sync_copy(bits_v, bits_hbm.at[pl.ds(g0, GPT)])


def kernel(tau, batch):
    emb, alpha, sigma = _embed(tau.reshape(G, 1))
    cond, bits = _sc_body(emb, batch.astype(jnp.int32))
    return cond, alpha, sigma, bits


# speedup vs baseline: 1.7191x; 1.7191x over previous
import functools

import jax
import jax.numpy as jnp
from jax import lax
from jax.experimental import pallas as pl
from jax.experimental.pallas import tpu as pltpu
from jax.experimental.pallas import tpu_sc as plsc

G = 4096
N = 100000
D = 128
NB = 8
HALF = D // 2

NC = 2
NS = 16
NW = NC * NS

CH = 80
SPAN = 3120
NCH_LO = SPAN // CH
NCH_HI = 41
SPAN_HI = NCH_HI * CH
NBUF = 4

CSPAN = 6256
CSPAN_HI = N - (NS - 1) * CSPAN
NVEC_LO = CSPAN // 16
NVEC_HI = CSPAN_HI // 16
GPT = G // NS


def _embed_body(tau_ref, emb_ref, alpha_ref, sigma_ref):
    t = tau_ref[...]
    k = lax.broadcasted_iota(jnp.int32, (1, HALF), 1).astype(jnp.float32)
    freqs = jnp.exp((-jnp.log(10000.0) / HALF) * k)
    args = t * freqs
    emb_ref[:, :HALF] = jnp.sin(args)
    emb_ref[:, HALF:] = jnp.cos(args)
    alpha_ref[...] = 1.0 - t
    sigma_ref[...] = t


_embed = pl.pallas_call(
    _embed_body,
    out_shape=(
        jax.ShapeDtypeStruct((G, D), jnp.float32),
        jax.ShapeDtypeStruct((G, 1), jnp.float32),
        jax.ShapeDtypeStruct((G, 1), jnp.float32),
    ),
)


_mesh = plsc.VectorSubcoreMesh(
    core_axis_name="c", subcore_axis_name="s", num_cores=NC, num_subcores=NS
)


@functools.partial(
    pl.kernel,
    out_type=(
        jax.ShapeDtypeStruct((N, D), jnp.float32),
        jax.ShapeDtypeStruct((G, NB), jnp.float32),
    ),
    mesh=_mesh,
    compiler_params=pltpu.CompilerParams(needs_layout_passes=False),
    scratch_types=(
        pltpu.VMEM((SPAN_HI,), jnp.int32),
        pltpu.VMEM((NBUF, CH, D), jnp.float32),
        pltpu.VMEM((CSPAN + 16,), jnp.int32),
        pltpu.VMEM((G,), jnp.int32),
        pltpu.VMEM((NS * GPT,), jnp.int32),
        pltpu.VMEM((GPT,), jnp.int32),
        pltpu.VMEM((GPT, NB), jnp.float32),
        pltpu.VMEM_SHARED((NS, G), jnp.int32),
        pltpu.SemaphoreType.DMA((NBUF,)),
        pltpu.SemaphoreType.DMA((NBUF,)),
    ),
)
def _sc_body(emb_hbm, batch_hbm, cond_hbm, bits_hbm,
             idx_all_v, rows_v, ext_all_v, pcount_v, ptmp_v, csum_v, bits_v,
             shared, gsem, wsem):
    cid = lax.axis_index("c")
    sid = lax.axis_index("s")
    wid = sid * NC + cid

    zeros16 = jnp.zeros((16,), jnp.int32)
    iota16 = lax.iota(jnp.int32, 16)

    @pl.when(cid == 0)
    def _counts():
        def zero_body(i, _):
            pcount_v[pl.ds(i * 16, 16)] = zeros16
            return _
        lax.fori_loop(0, G // 16, zero_body, None)

        cbase = sid * CSPAN
        last_tile = sid == NS - 1

        @pl.when(jnp.logical_not(last_tile))
        def _():
            pltpu.sync_copy(batch_hbm.at[pl.ds(cbase, CSPAN + 8)],
                            ext_all_v.at[pl.ds(0, CSPAN + 8)])

        @pl.when(last_tile)
        def _():
            pltpu.sync_copy(batch_hbm.at[pl.ds(cbase, CSPAN_HI)],
                            ext_all_v.at[pl.ds(0, CSPAN_HI)])
            ext_all_v[pl.ds(CSPAN_HI, 16)] = zeros16 - 1

        nvec = jnp.where(last_tile, NVEC_HI, NVEC_LO)

        def count_vec(v, _):
            j0 = v * 16
            cur = ext_all_v[pl.ds(j0, 16)]
            nxt = ext_all_v[pl.ds(j0 + 1, 16)]
            m = cur != nxt
            gi = (cbase + j0 + 1) + iota16
            plsc.addupdate_scatter(pcount_v, [cur], gi, mask=m)
            plsc.addupdate_scatter(pcount_v, [nxt], zeros16 - gi,
                                   mask=m & (nxt >= 0))
            return _
        lax.fori_loop(0, nvec, count_vec, None)

        pltpu.sync_copy(pcount_v, shared.at[sid])
        plsc.subcore_barrier()

    base = wid * SPAN
    last_w = wid == NW - 1
    nch = jnp.where(last_w, NCH_HI, NCH_LO)

    @pl.when(jnp.logical_not(last_w))
    def _():
        pltpu.sync_copy(batch_hbm.at[pl.ds(base, SPAN)],
                        idx_all_v.at[pl.ds(0, SPAN)])

    @pl.when(last_w)
    def _():
        pltpu.sync_copy(batch_hbm.at[pl.ds(base, SPAN_HI)], idx_all_v)

    def pipe_body(k, _):
        b = lax.rem(k, NBUF)
        bp = lax.rem(k + (NBUF - 1), NBUF)

        @pl.when(k < nch)
        def _start():
            @pl.when(k >= NBUF)
            def _():
                pltpu.make_async_copy(rows_v.at[b],
                                      cond_hbm.at[pl.ds(0, CH)],
                                      wsem.at[b]).wait()
            pltpu.async_copy(emb_hbm.at[idx_all_v.at[pl.ds(k * CH, CH)]],
                             rows_v.at[b], gsem.at[b])

        @pl.when((k >= 1) & (k <= nch))
        def _finish():
            km = k - 1
            pltpu.make_async_copy(cond_hbm.at[pl.ds(0, CH)],
                                  rows_v.at[bp], gsem.at[bp]).wait()
            pltpu.async_copy(rows_v.at[bp],
                             cond_hbm.at[pl.ds(base + km * CH, CH)],
                             wsem.at[bp])
        return _
    lax.fori_loop(0, NCH_HI + 1, pipe_body, None)

    for b in range(NBUF):
        pltpu.make_async_copy(rows_v.at[b], cond_hbm.at[pl.ds(0, CH)],
                              wsem.at[b]).wait()

    @pl.when(cid == 0)
    def _bits():
        g0 = sid * GPT
        for p in range(NS):
            pltpu.sync_copy(shared.at[p, pl.ds(g0, GPT)],
                            ptmp_v.at[pl.ds(p * GPT, GPT)])
        for v in range(0, GPT, 16):
            acc = zeros16
            for p in range(NS):
                acc = acc + ptmp_v[pl.ds(p * GPT + v, 16)]
            csum_v[pl.ds(v, 16)] = acc
        for v in range(0, GPT, 16):
            cnt = csum_v[pl.ds(v, 16)]
            rows = v + iota16
            for b in range(NB):
                bit = ((cnt >> b) & 1).astype(jnp.float32)
                cols = jnp.full((16,), b, jnp.int32)
                plsc.store_scatter(bits_v, [rows, cols], bit)
        pltpu.sync_copy(bits_v, bits_hbm.at[pl.ds(g0, GPT)])


def kernel(tau, batch):
    emb, alpha, sigma = _embed(tau.reshape(G, 1))
    cond, bits = _sc_body(emb, batch.astype(jnp.int32))
    return cond, alpha, sigma, bits


# speedup vs baseline: 1.9080x; 1.1099x over previous
import functools

import jax
import jax.numpy as jnp
from jax import lax
from jax.experimental import pallas as pl
from jax.experimental.pallas import tpu as pltpu
from jax.experimental.pallas import tpu_sc as plsc

G = 4096
N = 100000
D = 128
NB = 8
HALF = D // 2

NC = 2
NS = 16
NW = NC * NS

CH = 80
SPAN = 3120
NCH_LO = SPAN // CH
NCH_HI = 41
SPAN_HI = NCH_HI * CH
NBUF = 6
GD = 2

CSPAN = 6240
CSPAN_HI = N - (NS - 1) * CSPAN
NV2_LO = CSPAN // 32
NV2_HI = CSPAN_HI // 32
GPT = G // NS


def _embed_body(tau_ref, emb_ref, alpha_ref, sigma_ref):
    t = tau_ref[...]
    k = lax.broadcasted_iota(jnp.int32, (1, HALF), 1).astype(jnp.float32)
    freqs = jnp.exp((-jnp.log(10000.0) / HALF) * k)
    args = t * freqs
    emb_ref[:, :HALF] = jnp.sin(args)
    emb_ref[:, HALF:] = jnp.cos(args)
    alpha_ref[...] = 1.0 - t
    sigma_ref[...] = t


_embed = pl.pallas_call(
    _embed_body,
    out_shape=(
        jax.ShapeDtypeStruct((G, D), jnp.float32),
        jax.ShapeDtypeStruct((G, 1), jnp.float32),
        jax.ShapeDtypeStruct((G, 1), jnp.float32),
    ),
)


_mesh = plsc.VectorSubcoreMesh(
    core_axis_name="c", subcore_axis_name="s", num_cores=NC, num_subcores=NS
)


@functools.partial(
    pl.kernel,
    out_type=(
        jax.ShapeDtypeStruct((N, D), jnp.float32),
        jax.ShapeDtypeStruct((G, NB), jnp.float32),
    ),
    mesh=_mesh,
    compiler_params=pltpu.CompilerParams(needs_layout_passes=False),
    scratch_types=(
        pltpu.VMEM((SPAN_HI,), jnp.int32),
        pltpu.VMEM((NBUF, CH, D), jnp.float32),
        pltpu.VMEM((CSPAN_HI + 16,), jnp.int32),
        pltpu.VMEM((G,), jnp.int32),
        pltpu.VMEM((NS * GPT,), jnp.int32),
        pltpu.VMEM((GPT,), jnp.int32),
        pltpu.VMEM((GPT, NB), jnp.float32),
        pltpu.VMEM_SHARED((NS, G), jnp.int32),
        pltpu.SemaphoreType.DMA((NBUF,)),
        pltpu.SemaphoreType.DMA((NBUF,)),
    ),
)
def _sc_body(emb_hbm, batch_hbm, cond_hbm, bits_hbm,
             idx_all_v, rows_v, ext_all_v, pcount_v, ptmp_v, csum_v, bits_v,
             shared, gsem, wsem):
    cid = lax.axis_index("c")
    sid = lax.axis_index("s")
    wid = sid * NC + cid

    zeros16 = jnp.zeros((16,), jnp.int32)
    iota16 = lax.iota(jnp.int32, 16)

    @pl.when(cid == 0)
    def _counts():
        def zero_body(i, _):
            for j in range(8):
                pcount_v[pl.ds(i * 128 + j * 16, 16)] = zeros16
            return _
        lax.fori_loop(0, G // 128, zero_body, None)

        cbase = sid * CSPAN
        last_tile = sid == NS - 1

        @pl.when(jnp.logical_not(last_tile))
        def _():
            pltpu.sync_copy(batch_hbm.at[pl.ds(cbase, CSPAN + 8)],
                            ext_all_v.at[pl.ds(0, CSPAN + 8)])

        @pl.when(last_tile)
        def _():
            pltpu.sync_copy(batch_hbm.at[pl.ds(cbase, CSPAN_HI)],
                            ext_all_v.at[pl.ds(0, CSPAN_HI)])
            ext_all_v[pl.ds(CSPAN_HI, 16)] = zeros16 - 1

        nv2 = jnp.where(last_tile, NV2_HI, NV2_LO)

        def count_vec(v, _):
            for u in range(2):
                j0 = v * 32 + u * 16
                cur = ext_all_v[pl.ds(j0, 16)]
                nxt = ext_all_v[pl.ds(j0 + 1, 16)]
                m = cur != nxt
                gi = (cbase + j0 + 1) + iota16
                plsc.addupdate_scatter(pcount_v, [cur], gi, mask=m)
                plsc.addupdate_scatter(pcount_v, [nxt], zeros16 - gi,
                                       mask=m & (nxt >= 0))
            return _
        lax.fori_loop(0, nv2, count_vec, None)

        pltpu.sync_copy(pcount_v, shared.at[sid])
        plsc.subcore_barrier()

    base = wid * SPAN
    last_w = wid == NW - 1
    nch = jnp.where(last_w, NCH_HI, NCH_LO)

    @pl.when(jnp.logical_not(last_w))
    def _():
        pltpu.sync_copy(batch_hbm.at[pl.ds(base, SPAN)],
                        idx_all_v.at[pl.ds(0, SPAN)])

    @pl.when(last_w)
    def _():
        pltpu.sync_copy(batch_hbm.at[pl.ds(base, SPAN_HI)], idx_all_v)

    def pipe_body(k, _):
        b = lax.rem(k, NBUF)
        bp = lax.rem(k + (NBUF - GD), NBUF)

        @pl.when(k < nch)
        def _start():
            @pl.when(k >= NBUF)
            def _():
                pltpu.make_async_copy(rows_v.at[b],
                                      cond_hbm.at[pl.ds(0, CH)],
                                      wsem.at[b]).wait()
            pltpu.async_copy(emb_hbm.at[idx_all_v.at[pl.ds(k * CH, CH)]],
                             rows_v.at[b], gsem.at[b])

        @pl.when((k >= GD) & (k - GD < nch))
        def _finish():
            km = k - GD
            pltpu.make_async_copy(cond_hbm.at[pl.ds(0, CH)],
                                  rows_v.at[bp], gsem.at[bp]).wait()
            pltpu.async_copy(rows_v.at[bp],
                             cond_hbm.at[pl.ds(base + km * CH, CH)],
                             wsem.at[bp])
        return _
    lax.fori_loop(0, NCH_HI + GD, pipe_body, None)

    for b in range(NBUF):
        pltpu.make_async_copy(rows_v.at[b], cond_hbm.at[pl.ds(0, CH)],
                              wsem.at[b]).wait()

    @pl.when(cid == 0)
    def _bits():
        g0 = sid * GPT
        for p in range(NS):
            pltpu.sync_copy(shared.at[p, pl.ds(g0, GPT)],
                            ptmp_v.at[pl.ds(p * GPT, GPT)])
        for v in range(0, GPT, 16):
            acc = zeros16
            for p in range(NS):
                acc = acc + ptmp_v[pl.ds(p * GPT + v, 16)]
            csum_v[pl.ds(v, 16)] = acc
        for v in range(0, GPT, 16):
            cnt = csum_v[pl.ds(v, 16)]
            rows = v + iota16
            for b in range(NB):
                bit = ((cnt >> b) & 1).astype(jnp.float32)
                cols = jnp.full((16,), b, jnp.int32)
                plsc.store_scatter(bits_v, [rows, cols], bit)
        pltpu.sync_copy(bits_v, bits_hbm.at[pl.ds(g0, GPT)])


def kernel(tau, batch):
    emb, alpha, sigma = _embed(tau.reshape(G, 1))
    cond, bits = _sc_body(emb, batch.astype(jnp.int32))
    return cond, alpha, sigma, bits


# speedup vs baseline: 3.7382x; 1.9593x over previous
import functools

import jax
import jax.numpy as jnp
from jax import lax
from jax.experimental import pallas as pl
from jax.experimental.pallas import tpu as pltpu
from jax.experimental.pallas import tpu_sc as plsc

G = 4096
N = 100000
D = 128
NB = 8
HALF = D // 2

NC = 2
NS = 16
NW = NC * NS

CH = 80
SPAN = 3120
NCH_LO = SPAN // CH
NCH_HI = 41
SPAN_HI = NCH_HI * CH
NBUF = 3
GD = 2

CSPAN = 6240
CSPAN_HI = N - (NS - 1) * CSPAN
NV2_LO = CSPAN // 32
NV2_HI = CSPAN_HI // 32
GPT = G // NS


def _embed_body(tau_ref, emb_ref, alpha_ref, sigma_ref):
    t = tau_ref[...]
    k = lax.broadcasted_iota(jnp.int32, (1, HALF), 1).astype(jnp.float32)
    freqs = jnp.exp((-jnp.log(10000.0) / HALF) * k)
    args = t * freqs
    emb_ref[:, :HALF] = jnp.sin(args)
    emb_ref[:, HALF:] = jnp.cos(args)
    alpha_ref[...] = 1.0 - t
    sigma_ref[...] = t


_embed = pl.pallas_call(
    _embed_body,
    out_shape=(
        jax.ShapeDtypeStruct((G, D), jnp.float32),
        jax.ShapeDtypeStruct((G, 1), jnp.float32),
        jax.ShapeDtypeStruct((G, 1), jnp.float32),
    ),
)


_mesh = plsc.VectorSubcoreMesh(
    core_axis_name="c", subcore_axis_name="s", num_cores=NC, num_subcores=NS
)


@functools.partial(
    pl.kernel,
    out_type=(
        jax.ShapeDtypeStruct((N, D), jnp.float32),
        jax.ShapeDtypeStruct((G, NB), jnp.float32),
    ),
    mesh=_mesh,
    compiler_params=pltpu.CompilerParams(needs_layout_passes=False),
    scratch_types=(
        pltpu.VMEM((SPAN_HI,), jnp.int32),
        pltpu.VMEM((NBUF, CH, D), jnp.float32),
        pltpu.VMEM((CSPAN_HI + 16,), jnp.int32),
        pltpu.VMEM((G,), jnp.int32),
        pltpu.VMEM((NS * GPT,), jnp.int32),
        pltpu.VMEM((GPT,), jnp.int32),
        pltpu.VMEM((GPT, NB), jnp.float32),
        pltpu.VMEM_SHARED((NS, G), jnp.int32),
        pltpu.VMEM_SHARED((G, D), jnp.float32),
        pltpu.SemaphoreType.DMA((NBUF,)),
        pltpu.SemaphoreType.DMA((NBUF,)),
    ),
)
def _sc_body(emb_hbm, batch_hbm, cond_hbm, bits_hbm,
             idx_all_v, rows_v, ext_all_v, pcount_v, ptmp_v, csum_v, bits_v,
             shared, tab_sh, gsem, wsem):
    cid = lax.axis_index("c")
    sid = lax.axis_index("s")
    wid = sid * NC + cid

    zeros16 = jnp.zeros((16,), jnp.int32)
    iota16 = lax.iota(jnp.int32, 16)

    pltpu.sync_copy(emb_hbm.at[pl.ds(sid * (G // NS), G // NS)],
                    tab_sh.at[pl.ds(sid * (G // NS), G // NS)])
    plsc.subcore_barrier()

    @pl.when(cid == 0)
    def _counts():
        def zero_body(i, _):
            for j in range(8):
                pcount_v[pl.ds(i * 128 + j * 16, 16)] = zeros16
            return _
        lax.fori_loop(0, G // 128, zero_body, None)

        cbase = sid * CSPAN
        last_tile = sid == NS - 1

        @pl.when(jnp.logical_not(last_tile))
        def _():
            pltpu.sync_copy(batch_hbm.at[pl.ds(cbase, CSPAN + 8)],
                            ext_all_v.at[pl.ds(0, CSPAN + 8)])

        @pl.when(last_tile)
        def _():
            pltpu.sync_copy(batch_hbm.at[pl.ds(cbase, CSPAN_HI)],
                            ext_all_v.at[pl.ds(0, CSPAN_HI)])
            ext_all_v[pl.ds(CSPAN_HI, 16)] = zeros16 - 1

        nv2 = jnp.where(last_tile, NV2_HI, NV2_LO)

        def count_vec(v, _):
            for u in range(2):
                j0 = v * 32 + u * 16
                cur = ext_all_v[pl.ds(j0, 16)]
                nxt = ext_all_v[pl.ds(j0 + 1, 16)]
                m = cur != nxt
                gi = (cbase + j0 + 1) + iota16
                plsc.addupdate_scatter(pcount_v, [cur], gi, mask=m)
                plsc.addupdate_scatter(pcount_v, [nxt], zeros16 - gi,
                                       mask=m & (nxt >= 0))
            return _
        lax.fori_loop(0, nv2, count_vec, None)

        pltpu.sync_copy(pcount_v, shared.at[sid])
        plsc.subcore_barrier()

    base = wid * SPAN
    last_w = wid == NW - 1
    nch = jnp.where(last_w, NCH_HI, NCH_LO)

    @pl.when(jnp.logical_not(last_w))
    def _():
        pltpu.sync_copy(batch_hbm.at[pl.ds(base, SPAN)],
                        idx_all_v.at[pl.ds(0, SPAN)])

    @pl.when(last_w)
    def _():
        pltpu.sync_copy(batch_hbm.at[pl.ds(base, SPAN_HI)], idx_all_v)

    def pipe_body(k, _):
        b = lax.rem(k, NBUF)
        bp = lax.rem(k + (NBUF - GD), NBUF)

        @pl.when(k < nch)
        def _start():
            @pl.when(k >= NBUF)
            def _():
                pltpu.make_async_copy(rows_v.at[b],
                                      cond_hbm.at[pl.ds(0, CH)],
                                      wsem.at[b]).wait()
            pltpu.async_copy(tab_sh.at[idx_all_v.at[pl.ds(k * CH, CH)]],
                             rows_v.at[b], gsem.at[b])

        @pl.when((k >= GD) & (k - GD < nch))
        def _finish():
            km = k - GD
            pltpu.make_async_copy(cond_hbm.at[pl.ds(0, CH)],
                                  rows_v.at[bp], gsem.at[bp]).wait()
            pltpu.async_copy(rows_v.at[bp],
                             cond_hbm.at[pl.ds(base + km * CH, CH)],
                             wsem.at[bp])
        return _
    lax.fori_loop(0, NCH_HI + GD, pipe_body, None)

    for b in range(NBUF):
        pltpu.make_async_copy(rows_v.at[b], cond_hbm.at[pl.ds(0, CH)],
                              wsem.at[b]).wait()

    @pl.when(cid == 0)
    def _bits():
        g0 = sid * GPT
        for p in range(NS):
            pltpu.sync_copy(shared.at[p, pl.ds(g0, GPT)],
                            ptmp_v.at[pl.ds(p * GPT, GPT)])
        for v in range(0, GPT, 16):
            acc = zeros16
            for p in range(NS):
                acc = acc + ptmp_v[pl.ds(p * GPT + v, 16)]
            csum_v[pl.ds(v, 16)] = acc
        for v in range(0, GPT, 16):
            cnt = csum_v[pl.ds(v, 16)]
            rows = v + iota16
            for b in range(NB):
                bit = ((cnt >> b) & 1).astype(jnp.float32)
                cols = jnp.full((16,), b, jnp.int32)
                plsc.store_scatter(bits_v, [rows, cols], bit)
        pltpu.sync_copy(bits_v, bits_hbm.at[pl.ds(g0, GPT)])


def kernel(tau, batch):
    emb, alpha, sigma = _embed(tau.reshape(G, 1))
    cond, bits = _sc_body(emb, batch.astype(jnp.int32))
    return cond, alpha, sigma, bits
